# baseline (speedup 1.0000x reference)
import functools

import jax
import jax.numpy as jnp
from jax import lax
from jax.experimental import pallas as pl
from jax.experimental.pallas import tpu as pltpu
from jax.experimental.pallas import tpu_sc as plsc

TINY = 1e-6
M = 128
N = 64
BATCH = 16384
NC, NS, L = 2, 16, 16
NW = NC * NS
RAYS_PER_W = BATCH // NW
G = 64
NBATCH = RAYS_PER_W // G
NGRP = G // L
DELTA = 4.0 / 127.0


def _body(o_hbm, d_hbm, w_hbm, pts_hbm, z_hbm, s_hbm,
          w_v, o_v, d_v, cdf_v, pts_v, z_v):
    wid = lax.axis_index("s") * NC + lax.axis_index("c")
    iota = lax.iota(jnp.int32, L)

    def batch_body(g, carry):
        base = wid * RAYS_PER_W + g * G
        pltpu.sync_copy(w_hbm.at[pl.ds(base, G)], w_v)
        pltpu.sync_copy(o_hbm.at[pl.ds(base, G)], o_v)
        pltpu.sync_copy(d_hbm.at[pl.ds(base, G)], d_v)

        totals = []
        recips = []
        rays = []
        ods = []
        for grp in range(NGRP):
            rvec = iota + grp * L
            rays.append(rvec)
            c = jnp.zeros((L,), jnp.float32)
            cg = cdf_v.at[grp]
            for m in range(M):
                wv = plsc.load_gather(w_v, [rvec, jnp.full((L,), m, jnp.int32)])
                c = c + (wv + TINY)
                cg[m, :] = c
            totals.append(c)
            recips.append(1.0 / c)
            comps = []
            for ref in (o_v, d_v):
                for comp in range(3):
                    comps.append(plsc.load_gather(
                        ref, [rvec, jnp.full((L,), comp, jnp.int32)]))
            ods.append(comps)

        def sample_body(n, carry):
            nf = jnp.full((L,), n, jnp.int32).astype(jnp.float32)
            u = nf * (1.0 / 63.0)
            nvec = jnp.full((L,), n, jnp.int32)
            for grp in range(NGRP):
                cg = cdf_v.at[grp]
                U = u * totals[grp]
                below = jnp.zeros((L,), jnp.int32)
                for step in (64, 32, 16, 8, 4, 2, 1):
                    cand = below + step
                    val = plsc.load_gather(cg, [cand - 1, iota])
                    below = jnp.where(val <= U, cand, below)
                cBraw = plsc.load_gather(cg, [jnp.maximum(below - 1, 0), iota])
                cB = jnp.where(below > 0, cBraw, 0.0)
                cA = plsc.load_gather(cg, [below, iota])
                recip = recips[grp]
                denom = (cA - cB) * recip
                denom = jnp.where(denom < TINY, 1.0, denom)
                t = (u - cB * recip) / denom
                bf = below.astype(jnp.float32)
                blo = jnp.clip(bf - 0.5, 0.0, 127.0)
                bhi = jnp.minimum(bf + 0.5, 127.0)
                samples = 2.0 + blo * DELTA + t * ((bhi - blo) * DELTA + TINY)
                rvec = rays[grp]
                plsc.store_scatter(z_v, [rvec, nvec], samples)
                ox, oy, oz, dx, dy, dz = ods[grp]
                for comp, (o_s, d_s) in enumerate(
                        ((ox, dx), (oy, dy), (oz, dz))):
                    cvec = jnp.full((L,), comp, jnp.int32)
                    plsc.store_scatter(
                        pts_v, [rvec, nvec, cvec], o_s + d_s * samples)
            return carry

        lax.fori_loop(0, N, sample_body, 0, unroll=2)

        pltpu.sync_copy(pts_v, pts_hbm.at[pl.ds(base, G)])
        pltpu.sync_copy(z_v, z_hbm.at[pl.ds(base, G)])
        pltpu.sync_copy(z_v, s_hbm.at[pl.ds(base, G)])
        return carry

    lax.fori_loop(0, NBATCH, batch_body, 0, unroll=False)


@jax.jit
def kernel(rays_o, rays_d, weights):
    mesh = plsc.VectorSubcoreMesh(core_axis_name="c", subcore_axis_name="s")
    f = pl.kernel(
        _body,
        out_type=(
            jax.ShapeDtypeStruct((BATCH, N, 3), jnp.float32),
            jax.ShapeDtypeStruct((BATCH, N), jnp.float32),
            jax.ShapeDtypeStruct((BATCH, N), jnp.float32),
        ),
        mesh=mesh,
        compiler_params=pltpu.CompilerParams(
            needs_layout_passes=False, use_tc_tiling_on_sc=False),
        scratch_types=[
            pltpu.VMEM((G, M), jnp.float32),
            pltpu.VMEM((G, 3), jnp.float32),
            pltpu.VMEM((G, 3), jnp.float32),
            pltpu.VMEM((NGRP, M, L), jnp.float32),
            pltpu.VMEM((G, N, 3), jnp.float32),
            pltpu.VMEM((G, N), jnp.float32),
        ],
    )
    pts, z, s = f(rays_o, rays_d, weights)
    return (pts, z, s)
